# Initial kernel scaffold; baseline (speedup 1.0000x reference)
import jax
import jax.numpy as jnp
from jax.experimental import pallas as pl


def kernel(z, rbf, idx_i, idx_j, node_table, W_rbf, W_edge, b_edge):
    raise NotImplementedError("write your pallas kernel here")


# speedup vs baseline: 4.4295x; 4.4295x over previous
import functools

import jax
import jax.numpy as jnp
from jax import lax
from jax.experimental import pallas as pl
from jax.experimental.pallas import tpu as pltpu
from jax.experimental.pallas import tpu_sc as plsc

N_NODES = 10000
N_EDGES = 320000
NODE_DIM = 128
EDGE_DIM = 128
N_RADIAL = 16
TABLE_ROWS = 100

_L = 16


def _sc_gather_body(nc, chunk, z_hbm, ii_hbm, ij_hbm, zi_hbm, zj_hbm,
                    idx_v, out_v, sem):
    wid = lax.axis_index("s") * nc + lax.axis_index("c")
    base = wid * chunk
    for src, dst in ((ii_hbm, zi_hbm), (ij_hbm, zj_hbm)):
        pltpu.sync_copy(src.at[pl.ds(base, chunk)], idx_v)
        pltpu.async_copy(z_hbm.at[idx_v], out_v, sem).wait()
        pltpu.sync_copy(out_v, dst.at[pl.ds(base, chunk)])


def _sc_gather(z, idx_i, idx_j):
    info = plsc.get_sparse_core_info()
    nc, ns = info.num_cores, info.num_subcores
    nw = nc * ns
    chunk = N_EDGES // nw
    mesh = plsc.VectorSubcoreMesh(core_axis_name="c", subcore_axis_name="s")
    f = pl.kernel(
        functools.partial(_sc_gather_body, nc, chunk),
        mesh=mesh,
        out_type=[
            jax.ShapeDtypeStruct((N_EDGES,), jnp.int32),
            jax.ShapeDtypeStruct((N_EDGES,), jnp.int32),
        ],
        scratch_types=[
            pltpu.VMEM((chunk,), jnp.int32),
            pltpu.VMEM((chunk,), jnp.int32),
            pltpu.SemaphoreType.DMA,
        ],
    )
    return f(z, idx_i, idx_j)


def _prep_kernel(ntp_ref, we_ref, wr_ref, t1_ref, t2_ref, wc_ref):
    ntp = ntp_ref[:]
    t1_ref[:] = jnp.dot(ntp, we_ref[0:NODE_DIM, :],
                        preferred_element_type=jnp.float32)
    t2_ref[:] = jnp.dot(ntp, we_ref[NODE_DIM:2 * NODE_DIM, :],
                        preferred_element_type=jnp.float32)
    wc_ref[:] = jnp.dot(wr_ref[:], we_ref[2 * NODE_DIM:, :],
                        preferred_element_type=jnp.float32)


def _prep(ntp, W_edge, W_rbf):
    return pl.pallas_call(
        _prep_kernel,
        in_specs=[
            pl.BlockSpec((NODE_DIM, NODE_DIM), lambda: (0, 0)),
            pl.BlockSpec((2 * NODE_DIM + EDGE_DIM, EDGE_DIM), lambda: (0, 0)),
            pl.BlockSpec((N_RADIAL, EDGE_DIM), lambda: (0, 0)),
        ],
        out_specs=[
            pl.BlockSpec((NODE_DIM, EDGE_DIM), lambda: (0, 0)),
            pl.BlockSpec((NODE_DIM, EDGE_DIM), lambda: (0, 0)),
            pl.BlockSpec((N_RADIAL, EDGE_DIM), lambda: (0, 0)),
        ],
        out_shape=[
            jax.ShapeDtypeStruct((NODE_DIM, EDGE_DIM), jnp.float32),
            jax.ShapeDtypeStruct((NODE_DIM, EDGE_DIM), jnp.float32),
            jax.ShapeDtypeStruct((N_RADIAL, EDGE_DIM), jnp.float32),
        ],
    )(ntp, W_edge, W_rbf)


_BE = 2560
_NB = N_EDGES // _BE


def _main_kernel(zj_ref, zi_ref, rbf_ref, t1_ref, t2_ref, wc_ref, b_ref,
                 out_ref):
    zj = zj_ref[0, 0, :]
    zi = zi_ref[0, 0, :]
    iota = lax.broadcasted_iota(jnp.int32, (NODE_DIM, _BE), 0)
    ohj = (iota == zj).astype(jnp.float32)
    ohi = (iota == zi).astype(jnp.float32)
    dn = (((0,), (0,)), ((), ()))
    g = lax.dot_general(ohj, t1_ref[:], dn, preferred_element_type=jnp.float32)
    g = g + lax.dot_general(ohi, t2_ref[:], dn,
                            preferred_element_type=jnp.float32)
    r = jnp.dot(rbf_ref[:], wc_ref[:], preferred_element_type=jnp.float32)
    x = g + r + b_ref[:]
    out_ref[:] = x * jax.nn.sigmoid(x)


def _main(zj3, zi3, rbf, t1, t2, wc, b_edge):
    return pl.pallas_call(
        _main_kernel,
        grid=(_NB,),
        in_specs=[
            pl.BlockSpec((1, 1, _BE), lambda e: (e, 0, 0)),
            pl.BlockSpec((1, 1, _BE), lambda e: (e, 0, 0)),
            pl.BlockSpec((_BE, N_RADIAL), lambda e: (e, 0)),
            pl.BlockSpec((NODE_DIM, EDGE_DIM), lambda e: (0, 0)),
            pl.BlockSpec((NODE_DIM, EDGE_DIM), lambda e: (0, 0)),
            pl.BlockSpec((N_RADIAL, EDGE_DIM), lambda e: (0, 0)),
            pl.BlockSpec((EDGE_DIM,), lambda e: (0,)),
        ],
        out_specs=pl.BlockSpec((_BE, EDGE_DIM), lambda e: (e, 0)),
        out_shape=jax.ShapeDtypeStruct((N_EDGES, EDGE_DIM), jnp.float32),
    )(zj3, zi3, rbf, t1, t2, wc, b_edge)


def kernel(z, rbf, idx_i, idx_j, node_table, W_rbf, W_edge, b_edge):
    z = z.astype(jnp.int32)
    idx_i = idx_i.astype(jnp.int32)
    idx_j = idx_j.astype(jnp.int32)
    zi, zj = _sc_gather(z, idx_i, idx_j)
    ntp = jnp.pad(node_table, ((0, NODE_DIM - TABLE_ROWS), (0, 0)))
    t1, t2, wc = _prep(ntp, W_edge, W_rbf)
    zj3 = zj.reshape(_NB, 1, _BE)
    zi3 = zi.reshape(_NB, 1, _BE)
    return _main(zj3, zi3, rbf, t1, t2, wc, b_edge)


# speedup vs baseline: 5.1671x; 1.1665x over previous
import functools

import jax
import jax.numpy as jnp
from jax import lax
from jax.experimental import pallas as pl
from jax.experimental.pallas import tpu as pltpu
from jax.experimental.pallas import tpu_sc as plsc

N_NODES = 10000
N_EDGES = 320000
NODE_DIM = 128
EDGE_DIM = 128
N_RADIAL = 16
TABLE_ROWS = 100

_L = 16


def _sc_gather_body(nc, chunk, z_hbm, ii_hbm, ij_hbm, zi_hbm, zj_hbm,
                    z_sh, ii_v, ij_v, oi_v, oj_v, sem):
    s = lax.axis_index("s")
    wid = s * nc + lax.axis_index("c")
    base = wid * chunk

    @pl.when(s == 0)
    def _():
        pltpu.sync_copy(z_hbm, z_sh)

    plsc.subcore_barrier()
    pltpu.sync_copy(ii_hbm.at[pl.ds(base, chunk)], ii_v)
    pltpu.sync_copy(ij_hbm.at[pl.ds(base, chunk)], ij_v)
    ci = pltpu.async_copy(z_sh.at[ii_v], oi_v, sem)
    cj = pltpu.async_copy(z_sh.at[ij_v], oj_v, sem)
    ci.wait()
    cj.wait()
    pltpu.sync_copy(oi_v, zi_hbm.at[pl.ds(base, chunk)])
    pltpu.sync_copy(oj_v, zj_hbm.at[pl.ds(base, chunk)])


def _sc_gather(z, idx_i, idx_j):
    info = plsc.get_sparse_core_info()
    nc, ns = info.num_cores, info.num_subcores
    nw = nc * ns
    chunk = N_EDGES // nw
    mesh = plsc.VectorSubcoreMesh(core_axis_name="c", subcore_axis_name="s")
    f = pl.kernel(
        functools.partial(_sc_gather_body, nc, chunk),
        mesh=mesh,
        out_type=[
            jax.ShapeDtypeStruct((N_EDGES,), jnp.int32),
            jax.ShapeDtypeStruct((N_EDGES,), jnp.int32),
        ],
        scratch_types=[
            pltpu.VMEM_SHARED((N_NODES,), jnp.int32),
            pltpu.VMEM((chunk,), jnp.int32),
            pltpu.VMEM((chunk,), jnp.int32),
            pltpu.VMEM((chunk,), jnp.int32),
            pltpu.VMEM((chunk,), jnp.int32),
            pltpu.SemaphoreType.DMA,
        ],
    )
    return f(z, idx_i, idx_j)


def _prep_kernel(ntp_ref, we_ref, wr_ref, t1_ref, t2_ref, wc_ref):
    ntp = ntp_ref[:]
    t1_ref[:] = jnp.dot(ntp, we_ref[0:NODE_DIM, :],
                        preferred_element_type=jnp.float32)
    t2_ref[:] = jnp.dot(ntp, we_ref[NODE_DIM:2 * NODE_DIM, :],
                        preferred_element_type=jnp.float32)
    wc_ref[:] = jnp.dot(wr_ref[:], we_ref[2 * NODE_DIM:, :],
                        preferred_element_type=jnp.float32)


def _prep(ntp, W_edge, W_rbf):
    return pl.pallas_call(
        _prep_kernel,
        in_specs=[
            pl.BlockSpec((NODE_DIM, NODE_DIM), lambda: (0, 0)),
            pl.BlockSpec((2 * NODE_DIM + EDGE_DIM, EDGE_DIM), lambda: (0, 0)),
            pl.BlockSpec((N_RADIAL, EDGE_DIM), lambda: (0, 0)),
        ],
        out_specs=[
            pl.BlockSpec((NODE_DIM, EDGE_DIM), lambda: (0, 0)),
            pl.BlockSpec((NODE_DIM, EDGE_DIM), lambda: (0, 0)),
            pl.BlockSpec((N_RADIAL, EDGE_DIM), lambda: (0, 0)),
        ],
        out_shape=[
            jax.ShapeDtypeStruct((NODE_DIM, EDGE_DIM), jnp.float32),
            jax.ShapeDtypeStruct((NODE_DIM, EDGE_DIM), jnp.float32),
            jax.ShapeDtypeStruct((N_RADIAL, EDGE_DIM), jnp.float32),
        ],
    )(ntp, W_edge, W_rbf)


_BE = 2560
_NB = N_EDGES // _BE


def _main_kernel(zj_ref, zi_ref, rbf_ref, t1_ref, t2_ref, wc_ref, b_ref,
                 out_ref):
    zj = zj_ref[0, 0, :]
    zi = zi_ref[0, 0, :]
    iota = lax.broadcasted_iota(jnp.int32, (NODE_DIM, _BE), 0)
    ohj = (iota == zj).astype(jnp.float32)
    ohi = (iota == zi).astype(jnp.float32)
    dn = (((0,), (0,)), ((), ()))
    g = lax.dot_general(ohj, t1_ref[:], dn, preferred_element_type=jnp.float32)
    g = g + lax.dot_general(ohi, t2_ref[:], dn,
                            preferred_element_type=jnp.float32)
    r = jnp.dot(rbf_ref[:], wc_ref[:], preferred_element_type=jnp.float32)
    x = g + r + b_ref[:]
    out_ref[:] = x * jax.nn.sigmoid(x)


def _main(zj3, zi3, rbf, t1, t2, wc, b_edge):
    return pl.pallas_call(
        _main_kernel,
        grid=(_NB,),
        in_specs=[
            pl.BlockSpec((1, 1, _BE), lambda e: (e, 0, 0)),
            pl.BlockSpec((1, 1, _BE), lambda e: (e, 0, 0)),
            pl.BlockSpec((_BE, N_RADIAL), lambda e: (e, 0)),
            pl.BlockSpec((NODE_DIM, EDGE_DIM), lambda e: (0, 0)),
            pl.BlockSpec((NODE_DIM, EDGE_DIM), lambda e: (0, 0)),
            pl.BlockSpec((N_RADIAL, EDGE_DIM), lambda e: (0, 0)),
            pl.BlockSpec((EDGE_DIM,), lambda e: (0,)),
        ],
        out_specs=pl.BlockSpec((_BE, EDGE_DIM), lambda e: (e, 0)),
        out_shape=jax.ShapeDtypeStruct((N_EDGES, EDGE_DIM), jnp.float32),
    )(zj3, zi3, rbf, t1, t2, wc, b_edge)


def kernel(z, rbf, idx_i, idx_j, node_table, W_rbf, W_edge, b_edge):
    z = z.astype(jnp.int32)
    idx_i = idx_i.astype(jnp.int32)
    idx_j = idx_j.astype(jnp.int32)
    zi, zj = _sc_gather(z, idx_i, idx_j)
    ntp = jnp.pad(node_table, ((0, NODE_DIM - TABLE_ROWS), (0, 0)))
    t1, t2, wc = _prep(ntp, W_edge, W_rbf)
    zj3 = zj.reshape(_NB, 1, _BE)
    zi3 = zi.reshape(_NB, 1, _BE)
    return _main(zj3, zi3, rbf, t1, t2, wc, b_edge)


# speedup vs baseline: 6.1968x; 1.1993x over previous
import functools

import jax
import jax.numpy as jnp
from jax import lax
from jax.experimental import pallas as pl
from jax.experimental.pallas import tpu as pltpu
from jax.experimental.pallas import tpu_sc as plsc

N_NODES = 10000
N_EDGES = 320000
NODE_DIM = 128
EDGE_DIM = 128
N_RADIAL = 16
TABLE_ROWS = 100

_L = 16


def _sc_gather_body(nc, chunk, z_hbm, ii_hbm, ij_hbm, zi_hbm, zj_hbm,
                    z_sh, ii_v, ij_v, oi_v, oj_v, sem):
    s = lax.axis_index("s")
    wid = s * nc + lax.axis_index("c")
    base = wid * chunk

    @pl.when(s == 0)
    def _():
        pltpu.sync_copy(z_hbm, z_sh)

    plsc.subcore_barrier()
    pltpu.sync_copy(ii_hbm.at[pl.ds(base, chunk)], ii_v)
    pltpu.sync_copy(ij_hbm.at[pl.ds(base, chunk)], ij_v)
    ci = pltpu.async_copy(z_sh.at[ii_v], oi_v, sem)
    cj = pltpu.async_copy(z_sh.at[ij_v], oj_v, sem)
    ci.wait()
    cj.wait()
    pltpu.sync_copy(oi_v, zi_hbm.at[pl.ds(base, chunk)])
    pltpu.sync_copy(oj_v, zj_hbm.at[pl.ds(base, chunk)])


def _sc_gather(z, idx_i, idx_j):
    info = plsc.get_sparse_core_info()
    nc, ns = info.num_cores, info.num_subcores
    nw = nc * ns
    chunk = N_EDGES // nw
    mesh = plsc.VectorSubcoreMesh(core_axis_name="c", subcore_axis_name="s")
    f = pl.kernel(
        functools.partial(_sc_gather_body, nc, chunk),
        mesh=mesh,
        out_type=[
            jax.ShapeDtypeStruct((N_EDGES,), jnp.int32),
            jax.ShapeDtypeStruct((N_EDGES,), jnp.int32),
        ],
        scratch_types=[
            pltpu.VMEM_SHARED((N_NODES,), jnp.int32),
            pltpu.VMEM((chunk,), jnp.int32),
            pltpu.VMEM((chunk,), jnp.int32),
            pltpu.VMEM((chunk,), jnp.int32),
            pltpu.VMEM((chunk,), jnp.int32),
            pltpu.SemaphoreType.DMA,
        ],
    )
    return f(z, idx_i, idx_j)


def _prep_kernel(ntp_ref, we_ref, wr_ref, t12_ref, wc_ref):
    ntp = ntp_ref[:]
    t1 = jnp.dot(ntp, we_ref[0:NODE_DIM, :],
                 preferred_element_type=jnp.float32)
    t2 = jnp.dot(ntp, we_ref[NODE_DIM:2 * NODE_DIM, :],
                 preferred_element_type=jnp.float32)
    t12_ref[:] = jnp.concatenate([t1, t2], axis=0).astype(jnp.bfloat16)
    wc_ref[:] = jnp.dot(wr_ref[:], we_ref[2 * NODE_DIM:, :],
                        preferred_element_type=jnp.float32)


def _prep(ntp, W_edge, W_rbf):
    return pl.pallas_call(
        _prep_kernel,
        in_specs=[
            pl.BlockSpec((NODE_DIM, NODE_DIM), lambda: (0, 0)),
            pl.BlockSpec((2 * NODE_DIM + EDGE_DIM, EDGE_DIM), lambda: (0, 0)),
            pl.BlockSpec((N_RADIAL, EDGE_DIM), lambda: (0, 0)),
        ],
        out_specs=[
            pl.BlockSpec((2 * NODE_DIM, EDGE_DIM), lambda: (0, 0)),
            pl.BlockSpec((N_RADIAL, EDGE_DIM), lambda: (0, 0)),
        ],
        out_shape=[
            jax.ShapeDtypeStruct((2 * NODE_DIM, EDGE_DIM), jnp.bfloat16),
            jax.ShapeDtypeStruct((N_RADIAL, EDGE_DIM), jnp.float32),
        ],
    )(ntp, W_edge, W_rbf)


_BE = 6400
_NB = N_EDGES // _BE


def _main_kernel(zj_ref, zi_ref, rbf_ref, t12_ref, wc_ref, b_ref, out_ref):
    zj = zj_ref[0, 0, :]
    zi = zi_ref[0, 0, :]
    iota = lax.broadcasted_iota(jnp.int32, (NODE_DIM, _BE), 0)
    ohj = (iota == zj).astype(jnp.bfloat16)
    ohi = (iota == zi).astype(jnp.bfloat16)
    oh = jnp.concatenate([ohj, ohi], axis=0)
    dn = (((0,), (0,)), ((), ()))
    g = lax.dot_general(oh, t12_ref[:], dn, preferred_element_type=jnp.float32)
    r = jnp.dot(rbf_ref[:], wc_ref[:], preferred_element_type=jnp.float32)
    x = g + r + b_ref[:]
    out_ref[:] = x * jax.nn.sigmoid(x)


def _main(zj3, zi3, rbf, t12, wc, b_edge):
    return pl.pallas_call(
        _main_kernel,
        grid=(_NB,),
        in_specs=[
            pl.BlockSpec((1, 1, _BE), lambda e: (e, 0, 0)),
            pl.BlockSpec((1, 1, _BE), lambda e: (e, 0, 0)),
            pl.BlockSpec((_BE, N_RADIAL), lambda e: (e, 0)),
            pl.BlockSpec((2 * NODE_DIM, EDGE_DIM), lambda e: (0, 0)),
            pl.BlockSpec((N_RADIAL, EDGE_DIM), lambda e: (0, 0)),
            pl.BlockSpec((EDGE_DIM,), lambda e: (0,)),
        ],
        out_specs=pl.BlockSpec((_BE, EDGE_DIM), lambda e: (e, 0)),
        out_shape=jax.ShapeDtypeStruct((N_EDGES, EDGE_DIM), jnp.float32),
    )(zj3, zi3, rbf, t12, wc, b_edge)


def kernel(z, rbf, idx_i, idx_j, node_table, W_rbf, W_edge, b_edge):
    z = z.astype(jnp.int32)
    idx_i = idx_i.astype(jnp.int32)
    idx_j = idx_j.astype(jnp.int32)
    zi, zj = _sc_gather(z, idx_i, idx_j)
    ntp = jnp.pad(node_table, ((0, NODE_DIM - TABLE_ROWS), (0, 0)))
    t12, wc = _prep(ntp, W_edge, W_rbf)
    zj3 = zj.reshape(_NB, 1, _BE)
    zi3 = zi.reshape(_NB, 1, _BE)
    return _main(zj3, zi3, rbf, t12, wc, b_edge)


# speedup vs baseline: 6.2323x; 1.0057x over previous
import functools

import jax
import jax.numpy as jnp
from jax import lax
from jax.experimental import pallas as pl
from jax.experimental.pallas import tpu as pltpu
from jax.experimental.pallas import tpu_sc as plsc

N_NODES = 10000
N_EDGES = 320000
NODE_DIM = 128
EDGE_DIM = 128
N_RADIAL = 16
TABLE_ROWS = 100

_L = 16


def _sc_gather_body(nc, chunk, z_hbm, ii_hbm, ij_hbm, zi_hbm, zj_hbm,
                    z_sh, ii_v, ij_v, oi_v, oj_v, sem):
    s = lax.axis_index("s")
    wid = s * nc + lax.axis_index("c")
    base = wid * chunk

    @pl.when(s == 0)
    def _():
        pltpu.sync_copy(z_hbm, z_sh)

    plsc.subcore_barrier()
    pltpu.sync_copy(ii_hbm.at[pl.ds(base, chunk)], ii_v)
    pltpu.sync_copy(ij_hbm.at[pl.ds(base, chunk)], ij_v)
    ci = pltpu.async_copy(z_sh.at[ii_v], oi_v, sem)
    cj = pltpu.async_copy(z_sh.at[ij_v], oj_v, sem)
    ci.wait()
    cj.wait()
    pltpu.sync_copy(oi_v, zi_hbm.at[pl.ds(base, chunk)])
    pltpu.sync_copy(oj_v, zj_hbm.at[pl.ds(base, chunk)])


def _sc_gather(z, idx_i, idx_j):
    info = plsc.get_sparse_core_info()
    nc, ns = info.num_cores, info.num_subcores
    nw = nc * ns
    chunk = N_EDGES // nw
    mesh = plsc.VectorSubcoreMesh(core_axis_name="c", subcore_axis_name="s")
    f = pl.kernel(
        functools.partial(_sc_gather_body, nc, chunk),
        mesh=mesh,
        out_type=[
            jax.ShapeDtypeStruct((N_EDGES,), jnp.int32),
            jax.ShapeDtypeStruct((N_EDGES,), jnp.int32),
        ],
        scratch_types=[
            pltpu.VMEM_SHARED((N_NODES,), jnp.int32),
            pltpu.VMEM((chunk,), jnp.int32),
            pltpu.VMEM((chunk,), jnp.int32),
            pltpu.VMEM((chunk,), jnp.int32),
            pltpu.VMEM((chunk,), jnp.int32),
            pltpu.SemaphoreType.DMA,
        ],
    )
    return f(z, idx_i, idx_j)


_BE = 6400
_NB = N_EDGES // _BE


def _main_kernel(zj_ref, zi_ref, rbf_ref, ntp_ref, we_ref, wr_ref, b_ref,
                 out_ref, t12_s, wc_s):
    @pl.when(pl.program_id(0) == 0)
    def _fold_weights():
        ntp = ntp_ref[:]
        t1 = jnp.dot(ntp, we_ref[0:NODE_DIM, :],
                     preferred_element_type=jnp.float32)
        t2 = jnp.dot(ntp, we_ref[NODE_DIM:2 * NODE_DIM, :],
                     preferred_element_type=jnp.float32)
        t12_s[:] = jnp.concatenate([t1, t2], axis=0).astype(jnp.bfloat16)
        wc_s[:] = jnp.dot(wr_ref[:], we_ref[2 * NODE_DIM:, :],
                          preferred_element_type=jnp.float32)

    zj = zj_ref[0, 0, :]
    zi = zi_ref[0, 0, :]
    iota = lax.broadcasted_iota(jnp.int32, (NODE_DIM, _BE), 0)
    ohj = (iota == zj).astype(jnp.bfloat16)
    ohi = (iota == zi).astype(jnp.bfloat16)
    oh = jnp.concatenate([ohj, ohi], axis=0)
    dn = (((0,), (0,)), ((), ()))
    g = lax.dot_general(oh, t12_s[:], dn, preferred_element_type=jnp.float32)
    r = jnp.dot(rbf_ref[:], wc_s[:], preferred_element_type=jnp.float32)
    x = g + r + b_ref[:]
    out_ref[:] = x * jax.nn.sigmoid(x)


def _main(zj3, zi3, rbf, ntp, W_edge, W_rbf, b_edge):
    return pl.pallas_call(
        _main_kernel,
        grid=(_NB,),
        in_specs=[
            pl.BlockSpec((1, 1, _BE), lambda e: (e, 0, 0)),
            pl.BlockSpec((1, 1, _BE), lambda e: (e, 0, 0)),
            pl.BlockSpec((_BE, N_RADIAL), lambda e: (e, 0)),
            pl.BlockSpec((NODE_DIM, NODE_DIM), lambda e: (0, 0)),
            pl.BlockSpec((2 * NODE_DIM + EDGE_DIM, EDGE_DIM), lambda e: (0, 0)),
            pl.BlockSpec((N_RADIAL, EDGE_DIM), lambda e: (0, 0)),
            pl.BlockSpec((EDGE_DIM,), lambda e: (0,)),
        ],
        out_specs=pl.BlockSpec((_BE, EDGE_DIM), lambda e: (e, 0)),
        out_shape=jax.ShapeDtypeStruct((N_EDGES, EDGE_DIM), jnp.float32),
        scratch_shapes=[
            pltpu.VMEM((2 * NODE_DIM, EDGE_DIM), jnp.bfloat16),
            pltpu.VMEM((N_RADIAL, EDGE_DIM), jnp.float32),
        ],
    )(zj3, zi3, rbf, ntp, W_edge, W_rbf, b_edge)


def kernel(z, rbf, idx_i, idx_j, node_table, W_rbf, W_edge, b_edge):
    z = z.astype(jnp.int32)
    idx_i = idx_i.astype(jnp.int32)
    idx_j = idx_j.astype(jnp.int32)
    zi, zj = _sc_gather(z, idx_i, idx_j)
    ntp = jnp.pad(node_table, ((0, NODE_DIM - TABLE_ROWS), (0, 0)))
    zj3 = zj.reshape(_NB, 1, _BE)
    zi3 = zi.reshape(_NB, 1, _BE)
    return _main(zj3, zi3, rbf, ntp, W_edge, W_rbf, b_edge)


# speedup vs baseline: 9.1576x; 1.4694x over previous
import functools

import jax
import jax.numpy as jnp
from jax import lax
from jax.experimental import pallas as pl
from jax.experimental.pallas import tpu as pltpu
from jax.experimental.pallas import tpu_sc as plsc

N_NODES = 10000
N_EDGES = 320000
NODE_DIM = 128
EDGE_DIM = 128
N_RADIAL = 16
TABLE_ROWS = 100

_L = 16


def _sc_gather_body(nc, chunk, z_hbm, ii_hbm, ij_hbm, zi_hbm, zj_hbm,
                    z_sh, ii_v, ij_v, oi_v, oj_v, sem):
    s = lax.axis_index("s")
    wid = s * nc + lax.axis_index("c")
    base = wid * chunk

    @pl.when(s == 0)
    def _():
        pltpu.sync_copy(z_hbm, z_sh)

    plsc.subcore_barrier()
    pltpu.sync_copy(ii_hbm.at[pl.ds(base, chunk)], ii_v)
    pltpu.sync_copy(ij_hbm.at[pl.ds(base, chunk)], ij_v)
    ci = pltpu.async_copy(z_sh.at[ii_v], oi_v, sem)
    cj = pltpu.async_copy(z_sh.at[ij_v], oj_v, sem)
    ci.wait()
    cj.wait()
    pltpu.sync_copy(oi_v, zi_hbm.at[pl.ds(base, chunk)])
    pltpu.sync_copy(oj_v, zj_hbm.at[pl.ds(base, chunk)])


def _sc_gather(z, idx_i, idx_j):
    info = plsc.get_sparse_core_info()
    nc, ns = info.num_cores, info.num_subcores
    nw = nc * ns
    chunk = N_EDGES // nw
    mesh = plsc.VectorSubcoreMesh(core_axis_name="c", subcore_axis_name="s")
    f = pl.kernel(
        functools.partial(_sc_gather_body, nc, chunk),
        mesh=mesh,
        out_type=[
            jax.ShapeDtypeStruct((N_EDGES,), jnp.int32),
            jax.ShapeDtypeStruct((N_EDGES,), jnp.int32),
        ],
        scratch_types=[
            pltpu.VMEM_SHARED((N_NODES,), jnp.int32),
            pltpu.VMEM((chunk,), jnp.int32),
            pltpu.VMEM((chunk,), jnp.int32),
            pltpu.VMEM((chunk,), jnp.int32),
            pltpu.VMEM((chunk,), jnp.int32),
            pltpu.SemaphoreType.DMA,
        ],
    )
    return f(z, idx_i, idx_j)


_BE = 6400
_NB = N_EDGES // _BE


def _main_kernel(zj_ref, zi_ref, rbf_ref, ntp_ref, we_ref, wr_ref, b_ref,
                 out_ref, t12_s, wc_s):
    @pl.when(pl.program_id(0) == 0)
    def _fold_weights():
        ntp = ntp_ref[:]
        t1 = jnp.dot(ntp, we_ref[0:NODE_DIM, :],
                     preferred_element_type=jnp.float32)
        t2 = jnp.dot(ntp, we_ref[NODE_DIM:2 * NODE_DIM, :],
                     preferred_element_type=jnp.float32)
        t12_s[:] = jnp.concatenate([t1, t2], axis=0).astype(jnp.bfloat16)
        wc_s[:] = jnp.dot(wr_ref[:], we_ref[2 * NODE_DIM:, :],
                          preferred_element_type=jnp.float32)

    zj = zj_ref[0, 0, :]
    zi = zi_ref[0, 0, :]
    iota = lax.broadcasted_iota(jnp.int32, (NODE_DIM, _BE), 0)
    ohj = (iota == zj).astype(jnp.bfloat16)
    ohi = (iota == zi).astype(jnp.bfloat16)
    oh = jnp.concatenate([ohj, ohi], axis=0)
    dn = (((0,), (0,)), ((), ()))
    g = lax.dot_general(oh, t12_s[:], dn, preferred_element_type=jnp.float32)
    r = lax.dot_general(rbf_ref[:], wc_s[:], dn,
                        preferred_element_type=jnp.float32)
    x = g + r + b_ref[:]
    out_ref[:] = x * jax.nn.sigmoid(x)


def _main(zj3, zi3, rbf_t, ntp, W_edge, W_rbf, b_edge):
    return pl.pallas_call(
        _main_kernel,
        grid=(_NB,),
        in_specs=[
            pl.BlockSpec((1, 1, _BE), lambda e: (e, 0, 0)),
            pl.BlockSpec((1, 1, _BE), lambda e: (e, 0, 0)),
            pl.BlockSpec((N_RADIAL, _BE), lambda e: (0, e)),
            pl.BlockSpec((NODE_DIM, NODE_DIM), lambda e: (0, 0)),
            pl.BlockSpec((2 * NODE_DIM + EDGE_DIM, EDGE_DIM), lambda e: (0, 0)),
            pl.BlockSpec((N_RADIAL, EDGE_DIM), lambda e: (0, 0)),
            pl.BlockSpec((EDGE_DIM,), lambda e: (0,)),
        ],
        out_specs=pl.BlockSpec((_BE, EDGE_DIM), lambda e: (e, 0)),
        out_shape=jax.ShapeDtypeStruct((N_EDGES, EDGE_DIM), jnp.float32),
        scratch_shapes=[
            pltpu.VMEM((2 * NODE_DIM, EDGE_DIM), jnp.bfloat16),
            pltpu.VMEM((N_RADIAL, EDGE_DIM), jnp.float32),
        ],
    )(zj3, zi3, rbf_t, ntp, W_edge, W_rbf, b_edge)


def kernel(z, rbf, idx_i, idx_j, node_table, W_rbf, W_edge, b_edge):
    z = z.astype(jnp.int32)
    idx_i = idx_i.astype(jnp.int32)
    idx_j = idx_j.astype(jnp.int32)
    zi, zj = _sc_gather(z, idx_i, idx_j)
    ntp = jnp.pad(node_table, ((0, NODE_DIM - TABLE_ROWS), (0, 0)))
    zj3 = zj.reshape(_NB, 1, _BE)
    zi3 = zi.reshape(_NB, 1, _BE)
    return _main(zj3, zi3, rbf.T, ntp, W_edge, W_rbf, b_edge)


# speedup vs baseline: 10.7958x; 1.1789x over previous
import functools

import jax
import jax.numpy as jnp
from jax import lax
from jax.experimental import pallas as pl
from jax.experimental.pallas import tpu as pltpu
from jax.experimental.pallas import tpu_sc as plsc

N_NODES = 10000
N_EDGES = 320000
NODE_DIM = 128
EDGE_DIM = 128
N_RADIAL = 16
TABLE_ROWS = 100

_L = 16


def _sc_gather_body(nc, chunk, z_hbm, ii_hbm, ij_hbm, zi_hbm, zj_hbm,
                    z_sh, ii_v, ij_v, oi_v, oj_v, sem):
    s = lax.axis_index("s")
    wid = s * nc + lax.axis_index("c")
    base = wid * chunk

    @pl.when(s == 0)
    def _():
        pltpu.sync_copy(z_hbm, z_sh)

    plsc.subcore_barrier()
    pltpu.sync_copy(ii_hbm.at[pl.ds(base, chunk)], ii_v)
    pltpu.sync_copy(ij_hbm.at[pl.ds(base, chunk)], ij_v)
    ci = pltpu.async_copy(z_sh.at[ii_v], oi_v, sem)
    cj = pltpu.async_copy(z_sh.at[ij_v], oj_v, sem)
    ci.wait()
    cj.wait()
    pltpu.sync_copy(oi_v, zi_hbm.at[pl.ds(base, chunk)])
    pltpu.sync_copy(oj_v, zj_hbm.at[pl.ds(base, chunk)])


def _sc_gather(z, idx_i, idx_j):
    info = plsc.get_sparse_core_info()
    nc, ns = info.num_cores, info.num_subcores
    nw = nc * ns
    chunk = N_EDGES // nw
    mesh = plsc.VectorSubcoreMesh(core_axis_name="c", subcore_axis_name="s")
    f = pl.kernel(
        functools.partial(_sc_gather_body, nc, chunk),
        mesh=mesh,
        out_type=[
            jax.ShapeDtypeStruct((N_EDGES,), jnp.int32),
            jax.ShapeDtypeStruct((N_EDGES,), jnp.int32),
        ],
        scratch_types=[
            pltpu.VMEM_SHARED((N_NODES,), jnp.int32),
            pltpu.VMEM((chunk,), jnp.int32),
            pltpu.VMEM((chunk,), jnp.int32),
            pltpu.VMEM((chunk,), jnp.int32),
            pltpu.VMEM((chunk,), jnp.int32),
            pltpu.SemaphoreType.DMA,
        ],
    )
    return f(z, idx_i, idx_j)


_BE = 6400
_NB = N_EDGES // _BE


def _main_kernel(zj_ref, zi_ref, rbf_ref, ntp_ref, we_ref, wr_ref, b_ref,
                 out_ref, t_s):
    @pl.when(pl.program_id(0) == 0)
    def _fold_weights():
        ntp = ntp_ref[:]
        t1 = jnp.dot(ntp, we_ref[0:NODE_DIM, :],
                     preferred_element_type=jnp.float32)
        t2 = jnp.dot(ntp, we_ref[NODE_DIM:2 * NODE_DIM, :],
                     preferred_element_type=jnp.float32)
        wc = jnp.dot(wr_ref[:], we_ref[2 * NODE_DIM:, :],
                     preferred_element_type=jnp.float32)
        t_s[:] = jnp.concatenate([t1, t2, wc], axis=0).astype(jnp.bfloat16)

    zj = zj_ref[0, 0, :]
    zi = zi_ref[0, 0, :]
    iota = lax.broadcasted_iota(jnp.int32, (NODE_DIM, _BE), 0)
    ohj = (iota == zj).astype(jnp.bfloat16)
    ohi = (iota == zi).astype(jnp.bfloat16)
    a = jnp.concatenate([ohj, ohi, rbf_ref[:].astype(jnp.bfloat16)], axis=0)
    dn = (((0,), (0,)), ((), ()))
    x = lax.dot_general(a, t_s[:], dn, preferred_element_type=jnp.float32)
    x = x + b_ref[:]
    out_ref[:] = x * jax.nn.sigmoid(x)


def _main(zj3, zi3, rbf_t, ntp, W_edge, W_rbf, b_edge):
    return pl.pallas_call(
        _main_kernel,
        grid=(_NB,),
        in_specs=[
            pl.BlockSpec((1, 1, _BE), lambda e: (e, 0, 0)),
            pl.BlockSpec((1, 1, _BE), lambda e: (e, 0, 0)),
            pl.BlockSpec((N_RADIAL, _BE), lambda e: (0, e)),
            pl.BlockSpec((NODE_DIM, NODE_DIM), lambda e: (0, 0)),
            pl.BlockSpec((2 * NODE_DIM + EDGE_DIM, EDGE_DIM), lambda e: (0, 0)),
            pl.BlockSpec((N_RADIAL, EDGE_DIM), lambda e: (0, 0)),
            pl.BlockSpec((EDGE_DIM,), lambda e: (0,)),
        ],
        out_specs=pl.BlockSpec((_BE, EDGE_DIM), lambda e: (e, 0)),
        out_shape=jax.ShapeDtypeStruct((N_EDGES, EDGE_DIM), jnp.float32),
        scratch_shapes=[
            pltpu.VMEM((2 * NODE_DIM + N_RADIAL, EDGE_DIM), jnp.bfloat16),
        ],
    )(zj3, zi3, rbf_t, ntp, W_edge, W_rbf, b_edge)


def kernel(z, rbf, idx_i, idx_j, node_table, W_rbf, W_edge, b_edge):
    z = z.astype(jnp.int32)
    idx_i = idx_i.astype(jnp.int32)
    idx_j = idx_j.astype(jnp.int32)
    zi, zj = _sc_gather(z, idx_i, idx_j)
    ntp = jnp.pad(node_table, ((0, NODE_DIM - TABLE_ROWS), (0, 0)))
    zj3 = zj.reshape(_NB, 1, _BE)
    zi3 = zi.reshape(_NB, 1, _BE)
    return _main(zj3, zi3, rbf.T, ntp, W_edge, W_rbf, b_edge)


# speedup vs baseline: 11.2336x; 1.0406x over previous
import functools

import jax
import jax.numpy as jnp
from jax import lax
from jax.experimental import pallas as pl
from jax.experimental.pallas import tpu as pltpu
from jax.experimental.pallas import tpu_sc as plsc

N_NODES = 10000
N_EDGES = 320000
NODE_DIM = 128
EDGE_DIM = 128
N_RADIAL = 16
TABLE_ROWS = 100
CLS = 104

_L = 16


def _sc_gather_body(nc, chunk, z_hbm, ii_hbm, ij_hbm, zi_hbm, zj_hbm,
                    z_sh, ii_v, ij_v, oi_v, oj_v, sem):
    s = lax.axis_index("s")
    wid = s * nc + lax.axis_index("c")
    base = wid * chunk

    @pl.when(s == 0)
    def _():
        pltpu.sync_copy(z_hbm, z_sh)

    plsc.subcore_barrier()
    pltpu.sync_copy(ii_hbm.at[pl.ds(base, chunk)], ii_v)
    pltpu.sync_copy(ij_hbm.at[pl.ds(base, chunk)], ij_v)
    ci = pltpu.async_copy(z_sh.at[ii_v], oi_v, sem)
    cj = pltpu.async_copy(z_sh.at[ij_v], oj_v, sem)
    ci.wait()
    cj.wait()
    pltpu.sync_copy(oi_v, zi_hbm.at[pl.ds(base, chunk)])
    pltpu.sync_copy(oj_v, zj_hbm.at[pl.ds(base, chunk)])


def _sc_gather(z, idx_i, idx_j):
    info = plsc.get_sparse_core_info()
    nc, ns = info.num_cores, info.num_subcores
    nw = nc * ns
    chunk = N_EDGES // nw
    mesh = plsc.VectorSubcoreMesh(core_axis_name="c", subcore_axis_name="s")
    f = pl.kernel(
        functools.partial(_sc_gather_body, nc, chunk),
        mesh=mesh,
        out_type=[
            jax.ShapeDtypeStruct((N_EDGES,), jnp.int32),
            jax.ShapeDtypeStruct((N_EDGES,), jnp.int32),
        ],
        scratch_types=[
            pltpu.VMEM_SHARED((N_NODES,), jnp.int32),
            pltpu.VMEM((chunk,), jnp.int32),
            pltpu.VMEM((chunk,), jnp.int32),
            pltpu.VMEM((chunk,), jnp.int32),
            pltpu.VMEM((chunk,), jnp.int32),
            pltpu.SemaphoreType.DMA,
        ],
    )
    return f(z, idx_i, idx_j)


_BE = 6400
_NB = N_EDGES // _BE


def _main_kernel(zj_ref, zi_ref, rbf_ref, ntp_ref, we_ref, wr_ref, b_ref,
                 out_ref, t_s):
    @pl.when(pl.program_id(0) == 0)
    def _fold_weights():
        ntp = ntp_ref[:]
        t1 = jnp.dot(ntp, we_ref[0:NODE_DIM, :],
                     preferred_element_type=jnp.float32)[0:CLS, :]
        t2 = jnp.dot(ntp, we_ref[NODE_DIM:2 * NODE_DIM, :],
                     preferred_element_type=jnp.float32)[0:CLS, :]
        wc = jnp.dot(wr_ref[:], we_ref[2 * NODE_DIM:, :],
                     preferred_element_type=jnp.float32)
        t_s[:] = jnp.concatenate([t1, t2, wc], axis=0).astype(jnp.bfloat16)

    zj = zj_ref[0, 0, :]
    zi = zi_ref[0, 0, :]
    iota = lax.broadcasted_iota(jnp.int32, (CLS, _BE), 0)
    ohj = (iota == zj).astype(jnp.bfloat16)
    ohi = (iota == zi).astype(jnp.bfloat16)
    a = jnp.concatenate([ohj, ohi, rbf_ref[:].astype(jnp.bfloat16)], axis=0)
    dn = (((0,), (0,)), ((), ()))
    x = lax.dot_general(a, t_s[:], dn, preferred_element_type=jnp.float32)
    x = x + b_ref[:]
    out_ref[:] = x * jax.nn.sigmoid(x)


def _main(zj3, zi3, rbf_t, ntp, W_edge, W_rbf, b_edge):
    return pl.pallas_call(
        _main_kernel,
        grid=(_NB,),
        in_specs=[
            pl.BlockSpec((1, 1, _BE), lambda e: (e, 0, 0)),
            pl.BlockSpec((1, 1, _BE), lambda e: (e, 0, 0)),
            pl.BlockSpec((N_RADIAL, _BE), lambda e: (0, e)),
            pl.BlockSpec((NODE_DIM, NODE_DIM), lambda e: (0, 0)),
            pl.BlockSpec((2 * NODE_DIM + EDGE_DIM, EDGE_DIM), lambda e: (0, 0)),
            pl.BlockSpec((N_RADIAL, EDGE_DIM), lambda e: (0, 0)),
            pl.BlockSpec((EDGE_DIM,), lambda e: (0,)),
        ],
        out_specs=pl.BlockSpec((_BE, EDGE_DIM), lambda e: (e, 0)),
        out_shape=jax.ShapeDtypeStruct((N_EDGES, EDGE_DIM), jnp.float32),
        scratch_shapes=[
            pltpu.VMEM((2 * CLS + N_RADIAL, EDGE_DIM), jnp.bfloat16),
        ],
    )(zj3, zi3, rbf_t, ntp, W_edge, W_rbf, b_edge)


def kernel(z, rbf, idx_i, idx_j, node_table, W_rbf, W_edge, b_edge):
    z = z.astype(jnp.int32)
    idx_i = idx_i.astype(jnp.int32)
    idx_j = idx_j.astype(jnp.int32)
    zi, zj = _sc_gather(z, idx_i, idx_j)
    ntp = jnp.pad(node_table, ((0, NODE_DIM - TABLE_ROWS), (0, 0)))
    zj3 = zj.reshape(_NB, 1, _BE)
    zi3 = zi.reshape(_NB, 1, _BE)
    return _main(zj3, zi3, rbf.T, ntp, W_edge, W_rbf, b_edge)


# speedup vs baseline: 11.5810x; 1.0309x over previous
import functools

import jax
import jax.numpy as jnp
from jax import lax
from jax.experimental import pallas as pl
from jax.experimental.pallas import tpu as pltpu
from jax.experimental.pallas import tpu_sc as plsc

N_NODES = 10000
N_EDGES = 320000
NODE_DIM = 128
EDGE_DIM = 128
N_RADIAL = 16
TABLE_ROWS = 100
CLS = 104

_L = 16


def _sc_gather_body(nw, nb, be, z_hbm, ii_hbm, ij_hbm, zi_hbm, zj_hbm,
                    z_sh, ii_v, ij_v, oi_v, oj_v, sem):
    s = lax.axis_index("s")
    wid = s * (nw // 16) + lax.axis_index("c")

    @pl.when(s == 0)
    def _():
        pltpu.sync_copy(z_hbm, z_sh)

    plsc.subcore_barrier()
    for k in range((nb + nw - 1) // nw):
        u = wid + k * nw

        @pl.when(u < nb)
        def _():
            base = u * be
            pltpu.sync_copy(ii_hbm.at[pl.ds(base, be)], ii_v)
            pltpu.sync_copy(ij_hbm.at[pl.ds(base, be)], ij_v)
            ci = pltpu.async_copy(z_sh.at[ii_v], oi_v, sem)
            cj = pltpu.async_copy(z_sh.at[ij_v], oj_v, sem)
            ci.wait()
            cj.wait()
            pltpu.sync_copy(oi_v, zi_hbm.at[u, 0])
            pltpu.sync_copy(oj_v, zj_hbm.at[u, 0])


def _sc_gather(z, idx_i, idx_j, nb, be):
    info = plsc.get_sparse_core_info()
    nw = info.num_cores * info.num_subcores
    mesh = plsc.VectorSubcoreMesh(core_axis_name="c", subcore_axis_name="s")
    f = pl.kernel(
        functools.partial(_sc_gather_body, nw, nb, be),
        mesh=mesh,
        out_type=[
            jax.ShapeDtypeStruct((nb, 1, be), jnp.int32),
            jax.ShapeDtypeStruct((nb, 1, be), jnp.int32),
        ],
        scratch_types=[
            pltpu.VMEM_SHARED((N_NODES,), jnp.int32),
            pltpu.VMEM((be,), jnp.int32),
            pltpu.VMEM((be,), jnp.int32),
            pltpu.VMEM((be,), jnp.int32),
            pltpu.VMEM((be,), jnp.int32),
            pltpu.SemaphoreType.DMA,
        ],
    )
    return f(z, idx_i, idx_j)


_BE = 6400
_NB = N_EDGES // _BE


def _main_kernel(zj_ref, zi_ref, rbf_ref, ntp_ref, we_ref, wr_ref, b_ref,
                 out_ref, t_s):
    @pl.when(pl.program_id(0) == 0)
    def _fold_weights():
        ntp = ntp_ref[:]
        t1 = jnp.dot(ntp, we_ref[0:NODE_DIM, :],
                     preferred_element_type=jnp.float32)[0:CLS, :]
        t2 = jnp.dot(ntp, we_ref[NODE_DIM:2 * NODE_DIM, :],
                     preferred_element_type=jnp.float32)[0:CLS, :]
        wc = jnp.dot(wr_ref[:], we_ref[2 * NODE_DIM:, :],
                     preferred_element_type=jnp.float32)
        t_s[:] = jnp.concatenate([t1, t2, wc], axis=0).astype(jnp.bfloat16)

    zj = zj_ref[0, 0, :]
    zi = zi_ref[0, 0, :]
    iota = lax.broadcasted_iota(jnp.int32, (CLS, _BE), 0)
    ohj = (iota == zj).astype(jnp.bfloat16)
    ohi = (iota == zi).astype(jnp.bfloat16)
    a = jnp.concatenate([ohj, ohi, rbf_ref[:].astype(jnp.bfloat16)], axis=0)
    dn = (((0,), (0,)), ((), ()))
    x = lax.dot_general(a, t_s[:], dn, preferred_element_type=jnp.float32)
    x = x + b_ref[:]
    out_ref[:] = x * jax.nn.sigmoid(x)


def _main(zj3, zi3, rbf_t, ntp, W_edge, W_rbf, b_edge):
    return pl.pallas_call(
        _main_kernel,
        grid=(_NB,),
        in_specs=[
            pl.BlockSpec((1, 1, _BE), lambda e: (e, 0, 0)),
            pl.BlockSpec((1, 1, _BE), lambda e: (e, 0, 0)),
            pl.BlockSpec((N_RADIAL, _BE), lambda e: (0, e)),
            pl.BlockSpec((NODE_DIM, NODE_DIM), lambda e: (0, 0)),
            pl.BlockSpec((2 * NODE_DIM + EDGE_DIM, EDGE_DIM), lambda e: (0, 0)),
            pl.BlockSpec((N_RADIAL, EDGE_DIM), lambda e: (0, 0)),
            pl.BlockSpec((EDGE_DIM,), lambda e: (0,)),
        ],
        out_specs=pl.BlockSpec((_BE, EDGE_DIM), lambda e: (e, 0)),
        out_shape=jax.ShapeDtypeStruct((N_EDGES, EDGE_DIM), jnp.float32),
        scratch_shapes=[
            pltpu.VMEM((2 * CLS + N_RADIAL, EDGE_DIM), jnp.bfloat16),
        ],
    )(zj3, zi3, rbf_t, ntp, W_edge, W_rbf, b_edge)


def kernel(z, rbf, idx_i, idx_j, node_table, W_rbf, W_edge, b_edge):
    z = z.astype(jnp.int32)
    idx_i = idx_i.astype(jnp.int32)
    idx_j = idx_j.astype(jnp.int32)
    zi3, zj3 = _sc_gather(z, idx_i, idx_j, _NB, _BE)
    ntp = jnp.pad(node_table, ((0, NODE_DIM - TABLE_ROWS), (0, 0)))
    return _main(zj3, zi3, rbf.T, ntp, W_edge, W_rbf, b_edge)


# speedup vs baseline: 11.8170x; 1.0204x over previous
import functools

import jax
import jax.numpy as jnp
from jax import lax
from jax.experimental import pallas as pl
from jax.experimental.pallas import tpu as pltpu
from jax.experimental.pallas import tpu_sc as plsc

N_NODES = 10000
N_EDGES = 320000
NODE_DIM = 128
EDGE_DIM = 128
N_RADIAL = 16
TABLE_ROWS = 100
CLS = 104

_L = 16


def _sc_gather_body(nw, nb, be, u0, z_hbm, ii_hbm, ij_hbm, zi_hbm, zj_hbm,
                    z_sh, ii_v, ij_v, oi_v, oj_v, sem):
    s = lax.axis_index("s")
    wid = s * (nw // 16) + lax.axis_index("c")

    @pl.when(s == 0)
    def _():
        pltpu.sync_copy(z_hbm, z_sh)

    plsc.subcore_barrier()
    for k in range((nb + nw - 1) // nw):
        u = wid + k * nw

        @pl.when(u < nb)
        def _():
            base = (u0 + u) * be
            pltpu.sync_copy(ii_hbm.at[pl.ds(base, be)], ii_v)
            pltpu.sync_copy(ij_hbm.at[pl.ds(base, be)], ij_v)
            ci = pltpu.async_copy(z_sh.at[ii_v], oi_v, sem)
            cj = pltpu.async_copy(z_sh.at[ij_v], oj_v, sem)
            ci.wait()
            cj.wait()
            pltpu.sync_copy(oi_v, zi_hbm.at[u, 0])
            pltpu.sync_copy(oj_v, zj_hbm.at[u, 0])


def _sc_gather(z, idx_i, idx_j, nb, be, u0):
    info = plsc.get_sparse_core_info()
    nw = info.num_cores * info.num_subcores
    mesh = plsc.VectorSubcoreMesh(core_axis_name="c", subcore_axis_name="s")
    f = pl.kernel(
        functools.partial(_sc_gather_body, nw, nb, be, u0),
        mesh=mesh,
        out_type=[
            jax.ShapeDtypeStruct((nb, 1, be), jnp.int32),
            jax.ShapeDtypeStruct((nb, 1, be), jnp.int32),
        ],
        scratch_types=[
            pltpu.VMEM_SHARED((N_NODES,), jnp.int32),
            pltpu.VMEM((be,), jnp.int32),
            pltpu.VMEM((be,), jnp.int32),
            pltpu.VMEM((be,), jnp.int32),
            pltpu.VMEM((be,), jnp.int32),
            pltpu.SemaphoreType.DMA,
        ],
    )
    return f(z, idx_i, idx_j)


_BE = 6400
_NB = N_EDGES // _BE


def _main_kernel(zj_ref, zi_ref, rbf_ref, ntp_ref, we_ref, wr_ref, b_ref,
                 out_ref, t_s):
    @pl.when(pl.program_id(0) == 0)
    def _fold_weights():
        ntp = ntp_ref[:]
        t1 = jnp.dot(ntp, we_ref[0:NODE_DIM, :],
                     preferred_element_type=jnp.float32)[0:CLS, :]
        t2 = jnp.dot(ntp, we_ref[NODE_DIM:2 * NODE_DIM, :],
                     preferred_element_type=jnp.float32)[0:CLS, :]
        wc = jnp.dot(wr_ref[:], we_ref[2 * NODE_DIM:, :],
                     preferred_element_type=jnp.float32)
        t_s[:] = jnp.concatenate([t1, t2, wc], axis=0).astype(jnp.bfloat16)

    zj = zj_ref[0, 0, :]
    zi = zi_ref[0, 0, :]
    iota = lax.broadcasted_iota(jnp.int32, (CLS, _BE), 0)
    ohj = (iota == zj).astype(jnp.bfloat16)
    ohi = (iota == zi).astype(jnp.bfloat16)
    a = jnp.concatenate([ohj, ohi, rbf_ref[:].astype(jnp.bfloat16)], axis=0)
    dn = (((0,), (0,)), ((), ()))
    x = lax.dot_general(a, t_s[:], dn, preferred_element_type=jnp.float32)
    x = x + b_ref[:]
    out_ref[:] = x * jax.nn.sigmoid(x)


def _main_half(zj3, zi3, rbf_t, ntp, W_edge, W_rbf, b_edge, nb, blk0, prev):
    def body(zj_ref, zi_ref, rbf_ref, ntp_ref, we_ref, wr_ref, b_ref,
             *rest):
        if prev is not None:
            rest = rest[1:]
        out_ref, t_s = rest
        _main_kernel(zj_ref, zi_ref, rbf_ref, ntp_ref, we_ref, wr_ref, b_ref,
                     out_ref, t_s)

    in_specs = [
        pl.BlockSpec((1, 1, _BE), lambda e: (e, 0, 0)),
        pl.BlockSpec((1, 1, _BE), lambda e: (e, 0, 0)),
        pl.BlockSpec((N_RADIAL, _BE), lambda e: (0, e + blk0)),
        pl.BlockSpec((NODE_DIM, NODE_DIM), lambda e: (0, 0)),
        pl.BlockSpec((2 * NODE_DIM + EDGE_DIM, EDGE_DIM), lambda e: (0, 0)),
        pl.BlockSpec((N_RADIAL, EDGE_DIM), lambda e: (0, 0)),
        pl.BlockSpec((EDGE_DIM,), lambda e: (0,)),
    ]
    args = [zj3, zi3, rbf_t, ntp, W_edge, W_rbf, b_edge]
    kwargs = {}
    if prev is not None:
        in_specs.append(pl.BlockSpec(memory_space=pl.ANY))
        args.append(prev)
        kwargs["input_output_aliases"] = {7: 0}
    return pl.pallas_call(
        body,
        grid=(nb,),
        in_specs=in_specs,
        out_specs=pl.BlockSpec((_BE, EDGE_DIM), lambda e: (e + blk0, 0)),
        out_shape=jax.ShapeDtypeStruct((N_EDGES, EDGE_DIM), jnp.float32),
        scratch_shapes=[
            pltpu.VMEM((2 * CLS + N_RADIAL, EDGE_DIM), jnp.bfloat16),
        ],
        **kwargs,
    )(*args)


def kernel(z, rbf, idx_i, idx_j, node_table, W_rbf, W_edge, b_edge):
    z = z.astype(jnp.int32)
    idx_i = idx_i.astype(jnp.int32)
    idx_j = idx_j.astype(jnp.int32)
    ntp = jnp.pad(node_table, ((0, NODE_DIM - TABLE_ROWS), (0, 0)))
    rbf_t = rbf.T
    nb1 = _NB // 2
    nb2 = _NB - nb1
    zi_a, zj_a = _sc_gather(z, idx_i, idx_j, nb1, _BE, 0)
    zi_b, zj_b = _sc_gather(z, idx_i, idx_j, nb2, _BE, nb1)
    out1 = _main_half(zj_a, zi_a, rbf_t, ntp, W_edge, W_rbf, b_edge,
                      nb1, 0, None)
    return _main_half(zj_b, zi_b, rbf_t, ntp, W_edge, W_rbf, b_edge,
                      nb2, nb1, out1)
